# baseline (speedup 1.0000x reference)
import functools

import jax
import jax.numpy as jnp
from jax import lax
from jax.experimental import pallas as pl
from jax.experimental.pallas import tpu as pltpu
from jax.experimental.pallas import tpu_sc as plsc

N_NODES = 10000
N_EDGES = 320000
D_FEAT = 128

NUM_CORES = 2
NUM_SUBCORES = 16
NUM_TILES = NUM_CORES * NUM_SUBCORES

CHUNK = 80
EDGES_PER_TILE = N_EDGES // NUM_TILES
FULL_CHUNKS = EDGES_PER_TILE // CHUNK
TAIL = EDGES_PER_TILE - FULL_CHUNKS * CHUNK
TOTAL_CHUNKS = FULL_CHUNKS + (1 if TAIL else 0)
NBUF = 3
LOOKAHEAD = 2
MAIN_CHUNKS = ((TOTAL_CHUNKS - LOOKAHEAD) // NBUF) * NBUF
BLOCK_ROWS = 80
NUM_BLOCKS = N_NODES // BLOCK_ROWS


def _chunk_size(jj):
    return CHUNK if jj < FULL_CHUNKS else TAIL


TC_BLOCK_N = 1280
TC_GRID = (N_NODES + TC_BLOCK_N - 1) // TC_BLOCK_N


def _transpose_body(x_ref, o_ref):
    o_ref[...] = x_ref[...].T


def _to_node_major(state):
    return pl.pallas_call(
        _transpose_body,
        grid=(TC_GRID,),
        in_specs=[pl.BlockSpec((D_FEAT, TC_BLOCK_N), lambda i: (0, i))],
        out_specs=pl.BlockSpec((TC_BLOCK_N, D_FEAT), lambda i: (i, 0)),
        out_shape=jax.ShapeDtypeStruct((N_NODES, D_FEAT), jnp.float32),
    )(state)


def _combine_body(p_ref, o_ref):
    o_ref[...] = (p_ref[0] + p_ref[1]).T


def _combine(partials):
    return pl.pallas_call(
        _combine_body,
        grid=(TC_GRID,),
        in_specs=[pl.BlockSpec((2, TC_BLOCK_N, D_FEAT), lambda i: (0, i, 0))],
        out_specs=pl.BlockSpec((D_FEAT, TC_BLOCK_N), lambda i: (0, i)),
        out_shape=jax.ShapeDtypeStruct((D_FEAT, N_NODES), jnp.float32),
    )(partials)


def _sc_scatter_add(state_nd, src1d, dst1d):
    mesh = plsc.VectorSubcoreMesh(core_axis_name="c", subcore_axis_name="s")

    @functools.partial(
        pl.kernel,
        out_type=jax.ShapeDtypeStruct((NUM_CORES, N_NODES, D_FEAT), jnp.float32),
        mesh=mesh,
        scratch_types=[
            pltpu.VMEM((EDGES_PER_TILE,), jnp.int32),
            pltpu.VMEM((EDGES_PER_TILE,), jnp.int32),
        ] + [
            pltpu.VMEM((CHUNK, D_FEAT), jnp.float32) for _ in range(NBUF)
        ] + [
            pltpu.VMEM_SHARED((N_NODES, D_FEAT), jnp.float32),
        ] + [pltpu.SemaphoreType.DMA for _ in range(2 * NBUF + 2)],
    )
    def k(state_hbm, src_hbm, dst_hbm, out_hbm, src_v, dst_v, *rest):
        rows = rest[:NBUF]
        acc_sh = rest[NBUF]
        gsem = rest[NBUF + 1:2 * NBUF + 1]
        ssem = rest[2 * NBUF + 1:3 * NBUF + 1]
        isem = rest[3 * NBUF + 1:3 * NBUF + 3]
        c = lax.axis_index("c")
        s = lax.axis_index("s")
        wid = c * NUM_SUBCORES + s

        def buf(b, n):
            return rows[b] if n == CHUNK else rows[b].at[pl.ds(0, n)]

        def gather_start(jj, b, n=CHUNK):
            pltpu.async_copy(
                state_hbm.at[src_v.at[pl.ds(jj * CHUNK, n)]],
                buf(b, n), gsem[b])

        def gather_wait(jj, b, n=CHUNK):
            pltpu.make_async_copy(
                state_hbm.at[src_v.at[pl.ds(jj * CHUNK, n)]],
                buf(b, n), gsem[b]).wait()

        def scatter_start(jj, b, n=CHUNK):
            pltpu.async_copy(
                buf(b, n), acc_sh.at[dst_v.at[pl.ds(jj * CHUNK, n)]],
                ssem[b], add=True)

        def scatter_wait(jj, b, n=CHUNK):
            pltpu.make_async_copy(
                buf(b, n), acc_sh.at[dst_v.at[pl.ds(jj * CHUNK, n)]],
                ssem[b]).wait()

        e0 = wid * EDGES_PER_TILE
        idx_src = pltpu.async_copy(
            src_hbm.at[pl.ds(e0, EDGES_PER_TILE)], src_v, isem[0])
        idx_dst = pltpu.async_copy(
            dst_hbm.at[pl.ds(e0, EDGES_PER_TILE)], dst_v, isem[1])

        @pl.loop(0, BLOCK_ROWS)
        def _(r):
            @pl.loop(0, D_FEAT, step=16)
            def _(k16):
                rows[0][r, pl.ds(k16, 16)] = jnp.zeros((16,), jnp.float32)

        @pl.loop(0, NUM_BLOCKS)
        def _(b):
            @pl.when(lax.rem(b, NUM_SUBCORES) == s)
            def _():
                pltpu.sync_copy(
                    rows[0].at[pl.ds(0, BLOCK_ROWS)],
                    acc_sh.at[pl.ds(b * BLOCK_ROWS, BLOCK_ROWS)])

        idx_src.wait()
        idx_dst.wait()
        plsc.subcore_barrier()

        for b in range(LOOKAHEAD):
            gather_start(b, b)

        @pl.loop(0, MAIN_CHUNKS, step=NBUF)
        def _(j):
            for b in range(NBUF):
                jj = j + b
                gather_wait(jj, b)
                scatter_start(jj, b)
                bf = (b + LOOKAHEAD) % NBUF
                f = jj + LOOKAHEAD

                @pl.when(f >= NBUF)
                def _():
                    scatter_wait(f - NBUF, bf)

                gather_start(f, bf)

        for jj in range(MAIN_CHUNKS, TOTAL_CHUNKS):
            b = jj % NBUF
            gather_wait(jj, b, _chunk_size(jj))
            scatter_start(jj, b, _chunk_size(jj))
            f = jj + LOOKAHEAD
            if f < TOTAL_CHUNKS:
                bf = f % NBUF
                scatter_wait(f - NBUF, bf, _chunk_size(f - NBUF))
                gather_start(f, bf, _chunk_size(f))

        for jj in range(TOTAL_CHUNKS - NBUF, TOTAL_CHUNKS):
            scatter_wait(jj, jj % NBUF, _chunk_size(jj))

        plsc.subcore_barrier()

        @pl.loop(0, NUM_BLOCKS)
        def _(b):
            @pl.when(lax.rem(b, NUM_SUBCORES) == s)
            def _():
                r0 = b * BLOCK_ROWS
                pltpu.sync_copy(acc_sh.at[pl.ds(r0, BLOCK_ROWS)],
                                out_hbm.at[c].at[pl.ds(r0, BLOCK_ROWS)])

    return k(state_nd, src1d, dst1d)


def kernel(state, edge_index):
    state_nd = _to_node_major(state)
    partials = _sc_scatter_add(state_nd, edge_index[0], edge_index[1])
    return _combine(partials)


# speedup vs baseline: 1.0040x; 1.0040x over previous
import functools

import jax
import jax.numpy as jnp
from jax import lax
from jax.experimental import pallas as pl
from jax.experimental.pallas import tpu as pltpu
from jax.experimental.pallas import tpu_sc as plsc

N_NODES = 10000
N_EDGES = 320000
D_FEAT = 128

NUM_CORES = 2
NUM_SUBCORES = 16
NUM_TILES = NUM_CORES * NUM_SUBCORES

CHUNK = 80
EDGES_PER_TILE = N_EDGES // NUM_TILES
TOTAL_CHUNKS = EDGES_PER_TILE // CHUNK
NBUF = 4
LOOKAHEAD = 3
IDXBUF = 8
LOOKIDX = 6
MAIN_CHUNKS = ((TOTAL_CHUNKS - LOOKIDX - 1) // IDXBUF) * IDXBUF
BLOCK_ROWS = 80
NUM_BLOCKS = N_NODES // BLOCK_ROWS


def _transpose_body(x_ref, o_ref):
    o_ref[...] = x_ref[...].T


def _to_node_major(state):
    return pl.pallas_call(
        _transpose_body,
        out_shape=jax.ShapeDtypeStruct((N_NODES, D_FEAT), jnp.float32),
    )(state)


def _combine_body(p_ref, o_ref):
    o_ref[...] = (p_ref[0] + p_ref[1]).T


def _combine(partials):
    return pl.pallas_call(
        _combine_body,
        out_shape=jax.ShapeDtypeStruct((D_FEAT, N_NODES), jnp.float32),
    )(partials)


def _sc_scatter_add(state_nd, src1d, dst1d):
    mesh = plsc.VectorSubcoreMesh(core_axis_name="c", subcore_axis_name="s")

    @functools.partial(
        pl.kernel,
        out_type=jax.ShapeDtypeStruct((NUM_CORES, N_NODES, D_FEAT), jnp.float32),
        mesh=mesh,
        scratch_types=[
            pltpu.VMEM((CHUNK,), jnp.int32) for _ in range(2 * IDXBUF)
        ] + [
            pltpu.VMEM((CHUNK, D_FEAT), jnp.float32) for _ in range(NBUF)
        ] + [
            pltpu.VMEM_SHARED((N_NODES, D_FEAT), jnp.float32),
        ] + [pltpu.SemaphoreType.DMA for _ in range(2 * NBUF + 2 * IDXBUF)],
    )
    def k(state_hbm, src_hbm, dst_hbm, out_hbm, *rest):
        srcb = rest[:IDXBUF]
        dstb = rest[IDXBUF:2 * IDXBUF]
        rows = rest[2 * IDXBUF:2 * IDXBUF + NBUF]
        acc_sh = rest[2 * IDXBUF + NBUF]
        gsem = rest[2 * IDXBUF + NBUF + 1:2 * IDXBUF + 2 * NBUF + 1]
        ssem = rest[2 * IDXBUF + 2 * NBUF + 1:2 * IDXBUF + 3 * NBUF + 1]
        xsem = rest[2 * IDXBUF + 3 * NBUF + 1:3 * IDXBUF + 3 * NBUF + 1]
        ysem = rest[3 * IDXBUF + 3 * NBUF + 1:]
        c = lax.axis_index("c")
        s = lax.axis_index("s")
        wid = c * NUM_SUBCORES + s
        e0 = wid * EDGES_PER_TILE

        def idx_start(jj, q):
            pltpu.async_copy(
                src_hbm.at[pl.ds(e0 + jj * CHUNK, CHUNK)], srcb[q], xsem[q])
            pltpu.async_copy(
                dst_hbm.at[pl.ds(e0 + jj * CHUNK, CHUNK)], dstb[q], ysem[q])

        def idx_wait(jj, q):
            pltpu.make_async_copy(
                src_hbm.at[pl.ds(e0 + jj * CHUNK, CHUNK)], srcb[q],
                xsem[q]).wait()
            pltpu.make_async_copy(
                dst_hbm.at[pl.ds(e0 + jj * CHUNK, CHUNK)], dstb[q],
                ysem[q]).wait()

        def gather_start(b, q):
            pltpu.async_copy(state_hbm.at[srcb[q]], rows[b], gsem[b])

        def gather_wait(b, q):
            pltpu.make_async_copy(
                state_hbm.at[srcb[q]], rows[b], gsem[b]).wait()

        def scatter_start(b, q):
            pltpu.async_copy(
                rows[b], acc_sh.at[dstb[q]], ssem[b], add=True)

        def scatter_wait(b, q):
            pltpu.make_async_copy(
                rows[b], acc_sh.at[dstb[q]], ssem[b]).wait()

        for jj in range(LOOKIDX):
            idx_start(jj, jj % IDXBUF)

        @pl.loop(0, BLOCK_ROWS)
        def _(r):
            @pl.loop(0, D_FEAT, step=16)
            def _(k16):
                rows[0][r, pl.ds(k16, 16)] = jnp.zeros((16,), jnp.float32)

        @pl.loop(0, NUM_BLOCKS)
        def _(b):
            @pl.when(lax.rem(b, NUM_SUBCORES) == s)
            def _():
                pltpu.sync_copy(
                    rows[0].at[pl.ds(0, BLOCK_ROWS)],
                    acc_sh.at[pl.ds(b * BLOCK_ROWS, BLOCK_ROWS)])

        plsc.subcore_barrier()

        for jj in range(LOOKAHEAD):
            idx_wait(jj, jj % IDXBUF)
            gather_start(jj % NBUF, jj % IDXBUF)

        @pl.loop(0, MAIN_CHUNKS, step=IDXBUF)
        def _(j):
            for b in range(IDXBUF):
                jj = j + b
                gather_wait(b % NBUF, b)
                scatter_start(b % NBUF, b)
                f = jj + LOOKAHEAD
                bf = (b + LOOKAHEAD) % NBUF
                qf = (b + LOOKAHEAD) % IDXBUF

                if b + LOOKAHEAD >= NBUF:
                    scatter_wait(bf, (qf - NBUF) % IDXBUF)
                else:
                    @pl.when(f >= NBUF)
                    def _():
                        scatter_wait(bf, (qf - NBUF) % IDXBUF)

                idx_start(jj + LOOKIDX, (b + LOOKIDX) % IDXBUF)
                idx_wait(f, qf)
                gather_start(bf, qf)

        for jj in range(MAIN_CHUNKS, TOTAL_CHUNKS):
            b = jj % IDXBUF
            gather_wait(jj % NBUF, b)
            scatter_start(jj % NBUF, b)
            f = jj + LOOKAHEAD
            if f < TOTAL_CHUNKS:
                scatter_wait(f % NBUF, (f - NBUF) % IDXBUF)
                if jj + LOOKIDX < TOTAL_CHUNKS:
                    idx_start(jj + LOOKIDX, (jj + LOOKIDX) % IDXBUF)
                idx_wait(f, f % IDXBUF)
                gather_start(f % NBUF, f % IDXBUF)

        for jj in range(TOTAL_CHUNKS - NBUF, TOTAL_CHUNKS):
            scatter_wait(jj % NBUF, jj % IDXBUF)

        plsc.subcore_barrier()

        @pl.loop(0, NUM_BLOCKS)
        def _(b):
            @pl.when(lax.rem(b, NUM_SUBCORES) == s)
            def _():
                r0 = b * BLOCK_ROWS
                pltpu.sync_copy(acc_sh.at[pl.ds(r0, BLOCK_ROWS)],
                                out_hbm.at[c].at[pl.ds(r0, BLOCK_ROWS)])

    return k(state_nd, src1d, dst1d)


def kernel(state, edge_index):
    state_nd = _to_node_major(state)
    partials = _sc_scatter_add(state_nd, edge_index[0], edge_index[1])
    return _combine(partials)


# speedup vs baseline: 1.0300x; 1.0259x over previous
import functools

import jax
import jax.numpy as jnp
from jax import lax
from jax.experimental import pallas as pl
from jax.experimental.pallas import tpu as pltpu
from jax.experimental.pallas import tpu_sc as plsc

N_NODES = 10000
N_EDGES = 320000
D_FEAT = 128

NUM_CORES = 2
NUM_SUBCORES = 16
NUM_TILES = NUM_CORES * NUM_SUBCORES

CHUNK = 80
EDGES_PER_TILE = N_EDGES // NUM_TILES
TOTAL_CHUNKS = EDGES_PER_TILE // CHUNK
NBUF = 3
LOOKAHEAD = 2
MAIN_CHUNKS = ((TOTAL_CHUNKS - LOOKAHEAD) // NBUF) * NBUF
BLOCK_ROWS = 80
NUM_BLOCKS = N_NODES // BLOCK_ROWS


def _transpose_body(x_ref, o_ref):
    o_ref[...] = x_ref[...].T


def _to_node_major(state):
    return pl.pallas_call(
        _transpose_body,
        out_shape=jax.ShapeDtypeStruct((N_NODES, D_FEAT), jnp.float32),
    )(state)


def _combine_body(p_ref, o_ref):
    o_ref[...] = (p_ref[0] + p_ref[1]).T


def _combine(partials):
    return pl.pallas_call(
        _combine_body,
        out_shape=jax.ShapeDtypeStruct((D_FEAT, N_NODES), jnp.float32),
    )(partials)


def _sc_scatter_add(state_nd, src1d, dst1d):
    mesh = plsc.VectorSubcoreMesh(core_axis_name="c", subcore_axis_name="s")

    @functools.partial(
        pl.kernel,
        out_type=jax.ShapeDtypeStruct((NUM_CORES, N_NODES, D_FEAT), jnp.float32),
        mesh=mesh,
        scratch_types=[
            pltpu.VMEM((EDGES_PER_TILE,), jnp.int32),
            pltpu.VMEM((EDGES_PER_TILE,), jnp.int32),
        ] + [
            pltpu.VMEM((CHUNK, D_FEAT), jnp.float32) for _ in range(NBUF)
        ] + [
            pltpu.VMEM_SHARED((N_NODES, D_FEAT), jnp.float32),
        ] + [pltpu.SemaphoreType.DMA for _ in range(2 * NBUF + 2)],
    )
    def k(state_hbm, src_hbm, dst_hbm, out_hbm, src_v, dst_v, *rest):
        rows = rest[:NBUF]
        acc_sh = rest[NBUF]
        gsem = rest[NBUF + 1:2 * NBUF + 1]
        ssem = rest[2 * NBUF + 1:3 * NBUF + 1]
        isem = rest[3 * NBUF + 1:3 * NBUF + 3]
        c = lax.axis_index("c")
        s = lax.axis_index("s")
        wid = c * NUM_SUBCORES + s

        def gather_start(jj, b):
            pltpu.async_copy(
                state_hbm.at[src_v.at[pl.ds(jj * CHUNK, CHUNK)]],
                rows[b], gsem[b])

        def gather_wait(jj, b):
            pltpu.make_async_copy(
                state_hbm.at[src_v.at[pl.ds(jj * CHUNK, CHUNK)]],
                rows[b], gsem[b]).wait()

        def scatter_start(jj, b):
            pltpu.async_copy(
                rows[b], acc_sh.at[dst_v.at[pl.ds(jj * CHUNK, CHUNK)]],
                ssem[b], add=True)

        def scatter_wait(jj, b):
            pltpu.make_async_copy(
                rows[b], acc_sh.at[dst_v.at[pl.ds(jj * CHUNK, CHUNK)]],
                ssem[b]).wait()

        e0 = wid * EDGES_PER_TILE
        idx_src = pltpu.async_copy(
            src_hbm.at[pl.ds(e0, EDGES_PER_TILE)], src_v, isem[0])
        idx_dst = pltpu.async_copy(
            dst_hbm.at[pl.ds(e0, EDGES_PER_TILE)], dst_v, isem[1])

        @pl.loop(0, BLOCK_ROWS)
        def _(r):
            @pl.loop(0, D_FEAT, step=16)
            def _(k16):
                rows[0][r, pl.ds(k16, 16)] = jnp.zeros((16,), jnp.float32)

        idx_src.wait()

        for jj in range(LOOKAHEAD):
            gather_start(jj, jj + 1)

        @pl.loop(0, NUM_BLOCKS)
        def _(b):
            @pl.when(lax.rem(b, NUM_SUBCORES) == s)
            def _():
                pltpu.sync_copy(
                    rows[0].at[pl.ds(0, BLOCK_ROWS)],
                    acc_sh.at[pl.ds(b * BLOCK_ROWS, BLOCK_ROWS)])

        idx_dst.wait()
        plsc.subcore_barrier()

        @pl.loop(0, MAIN_CHUNKS, step=NBUF)
        def _(j):
            for b in range(NBUF):
                jj = j + b
                rb = (b + 1) % NBUF
                gather_wait(jj, rb)
                scatter_start(jj, rb)
                bf = (b + LOOKAHEAD + 1) % NBUF
                f = jj + LOOKAHEAD

                @pl.when(f >= NBUF)
                def _():
                    scatter_wait(f - NBUF, bf)

                gather_start(f, bf)

        for jj in range(MAIN_CHUNKS, TOTAL_CHUNKS):
            b = (jj + 1) % NBUF
            gather_wait(jj, b)
            scatter_start(jj, b)
            f = jj + LOOKAHEAD
            if f < TOTAL_CHUNKS:
                bf = (f + 1) % NBUF
                scatter_wait(f - NBUF, bf)
                gather_start(f, bf)

        for jj in range(TOTAL_CHUNKS - NBUF, TOTAL_CHUNKS):
            scatter_wait(jj, (jj + 1) % NBUF)

        plsc.subcore_barrier()

        def out_start(b, i):
            r0 = b * BLOCK_ROWS
            pltpu.async_copy(acc_sh.at[pl.ds(r0, BLOCK_ROWS)],
                             out_hbm.at[c].at[pl.ds(r0, BLOCK_ROWS)],
                             gsem[i % NBUF])

        def out_wait(b, i):
            r0 = b * BLOCK_ROWS
            pltpu.make_async_copy(acc_sh.at[pl.ds(r0, BLOCK_ROWS)],
                                  out_hbm.at[c].at[pl.ds(r0, BLOCK_ROWS)],
                                  gsem[i % NBUF]).wait()

        per_tile = (NUM_BLOCKS + NUM_SUBCORES - 1) // NUM_SUBCORES
        for i in range(per_tile):
            blk = i * NUM_SUBCORES + s
            if i >= NBUF:
                out_wait((i - NBUF) * NUM_SUBCORES + s, i - NBUF)
            if i < per_tile - 1:
                out_start(blk, i)
            else:
                @pl.when(blk < NUM_BLOCKS)
                def _():
                    out_start(blk, i)

        for i in range(per_tile - NBUF, per_tile):
            blk = i * NUM_SUBCORES + s
            if i < per_tile - 1:
                out_wait(blk, i)
            else:
                @pl.when(blk < NUM_BLOCKS)
                def _():
                    out_wait(blk, i)

    return k(state_nd, src1d, dst1d)


def kernel(state, edge_index):
    state_nd = _to_node_major(state)
    partials = _sc_scatter_add(state_nd, edge_index[0], edge_index[1])
    return _combine(partials)
